# initial kernel scaffold (unmeasured)
import functools

import jax
import jax.numpy as jnp
from jax import lax
from jax.experimental import pallas as pl
from jax.experimental.pallas import tpu as pltpu

N_DEV = 4
B, SQ, SKV_G, HQ_G, DH = 2, 512, 2048, 32, 64
H_LOC = HQ_G // N_DEV
SKV_LOC = SKV_G // N_DEV
DM = 768
DQ_LOC = H_LOC * DH


def kernel(x, Wq, K_ext, V_ext, Wo):
    def body(x_ref, wq_ref, k_ref, v_ref, wo_ref, out_ref,
             kb_ref, vb_ref, kg_ref, vg_ref, ar_ref,
             send_sems, recv_sems):
        me = lax.axis_index("i")

        barrier = pltpu.get_barrier_semaphore()
        for s in range(1, N_DEV):
            pl.semaphore_signal(
                barrier, inc=1,
                device_id=((me + s) % N_DEV,),
                device_id_type=pl.DeviceIdType.MESH,
            )
        pl.semaphore_wait(barrier, N_DEV - 1)

        kb_ref[...] = k_ref[...].astype(jnp.bfloat16)
        vb_ref[...] = v_ref[...].astype(jnp.bfloat16)

        my_h0 = me * H_LOC
        kg_ref[0] = pl.load(
            kb_ref, (slice(None), slice(None), pl.ds(my_h0, H_LOC), slice(None)))
        vg_ref[0] = pl.load(
            vb_ref, (slice(None), slice(None), pl.ds(my_h0, H_LOC), slice(None)))

        copies = []
        for s in range(1, N_DEV):
            peer = (me + s) % N_DEV
            ph0 = peer * H_LOC
            kcp = pltpu.make_async_remote_copy(
                src_ref=kb_ref.at[:, :, pl.ds(ph0, H_LOC), :],
                dst_ref=kg_ref.at[N_DEV - s],
                send_sem=send_sems.at[s - 1],
                recv_sem=recv_sems.at[s - 1],
                device_id=(peer,),
                device_id_type=pl.DeviceIdType.MESH,
            )
            vcp = pltpu.make_async_remote_copy(
                src_ref=vb_ref.at[:, :, pl.ds(ph0, H_LOC), :],
                dst_ref=vg_ref.at[N_DEV - s],
                send_sem=send_sems.at[3 + s - 1],
                recv_sem=recv_sems.at[3 + s - 1],
                device_id=(peer,),
                device_id_type=pl.DeviceIdType.MESH,
            )
            kcp.start()
            vcp.start()
            copies.append(kcp)
            copies.append(vcp)

        x2d = x_ref[...].astype(jnp.bfloat16).reshape(B * SQ, DM)
        wq = wq_ref[...].astype(jnp.bfloat16)
        qf = lax.dot_general(
            x2d, wq, (((1,), (0,)), ((), ())),
            preferred_element_type=jnp.float32,
        )
        qb = qf.astype(jnp.bfloat16).reshape(B, SQ, H_LOC, DH)

        qi = lax.broadcasted_iota(jnp.int32, (SQ, SKV_G), 0)
        col = lax.broadcasted_iota(jnp.int32, (SQ, SKV_G), 1)
        ki = ((me + col // SKV_LOC) % N_DEV) * SKV_LOC + col % SKV_LOC
        mask = (jnp.abs(qi - ki) <= 128) | (ki < 32) | (qi < 32)
        neg = jnp.float32(-1e9)

        for cp in copies:
            cp.wait_recv()

        ctx_rows = []
        for b in range(B):
            ctx_h = []
            for h in range(H_LOC):
                k2d = jnp.concatenate(
                    [kg_ref[t, b, :, h, :] for t in range(N_DEV)], axis=0)
                v2d = jnp.concatenate(
                    [vg_ref[t, b, :, h, :] for t in range(N_DEV)], axis=0)
                scores = lax.dot_general(
                    qb[b, :, h, :], k2d, (((1,), (1,)), ((), ())),
                    preferred_element_type=jnp.float32,
                ) * 0.125
                scores = jnp.where(mask, scores, neg)
                m = jnp.max(scores, axis=1, keepdims=True)
                w = jnp.exp(scores - m)
                w = w / jnp.sum(w, axis=1, keepdims=True)
                ctx_h.append(lax.dot_general(
                    w.astype(jnp.bfloat16), v2d, (((1,), (0,)), ((), ())),
                    preferred_element_type=jnp.float32,
                ))
            ctx_rows.append(jnp.concatenate(ctx_h, axis=1))
        ctx2d = jnp.concatenate(ctx_rows, axis=0).astype(jnp.bfloat16)

        wo = wo_ref[...].astype(jnp.bfloat16)
        partial = lax.dot_general(
            ctx2d, wo, (((1,), (0,)), ((), ())),
            preferred_element_type=jnp.float32,
        )
        ar_ref[0] = partial.astype(jnp.bfloat16)

        ar_copies = []
        for s in range(1, N_DEV):
            peer = (me + s) % N_DEV
            cp = pltpu.make_async_remote_copy(
                src_ref=ar_ref.at[0],
                dst_ref=ar_ref.at[N_DEV - s],
                send_sem=send_sems.at[6 + s - 1],
                recv_sem=recv_sems.at[6 + s - 1],
                device_id=(peer,),
                device_id_type=pl.DeviceIdType.MESH,
            )
            cp.start()
            ar_copies.append(cp)
        for cp in ar_copies:
            cp.wait_recv()

        acc = ar_ref[0].astype(jnp.float32)
        for t in range(1, N_DEV):
            acc = acc + ar_ref[t].astype(jnp.float32)
        out_ref[...] = acc.reshape(B, SQ, DM)

        for cp in copies:
            cp.wait_send()
        for cp in ar_copies:
            cp.wait_send()

    out_shape = jax.ShapeDtypeStruct((B, SQ, DM), jnp.float32)
    return pl.pallas_call(
        body,
        out_shape=out_shape,
        in_specs=[pl.BlockSpec(memory_space=pltpu.VMEM)] * 5,
        out_specs=pl.BlockSpec(memory_space=pltpu.VMEM),
        scratch_shapes=[
            pltpu.VMEM((B, SKV_LOC, HQ_G, DH), jnp.bfloat16),
            pltpu.VMEM((B, SKV_LOC, HQ_G, DH), jnp.bfloat16),
            pltpu.VMEM((N_DEV, B, SKV_LOC, H_LOC, DH), jnp.bfloat16),
            pltpu.VMEM((N_DEV, B, SKV_LOC, H_LOC, DH), jnp.bfloat16),
            pltpu.VMEM((N_DEV, B * SQ, DM), jnp.bfloat16),
            pltpu.SemaphoreType.DMA((9,)),
            pltpu.SemaphoreType.DMA((9,)),
        ],
        compiler_params=pltpu.CompilerParams(collective_id=0),
    )(x, Wq, K_ext, V_ext, Wo)


# baseline (device time: 138373 ns/iter reference)
import jax
import jax.numpy as jnp
from jax import lax
from jax.experimental import pallas as pl
from jax.experimental.pallas import tpu as pltpu

N_DEV = 4
B, SQ, SKV_G, HQ_G, DH = 2, 512, 2048, 32, 64
H_LOC = HQ_G // N_DEV
SKV_LOC = SKV_G // N_DEV
DM = 768
DQ_LOC = H_LOC * DH


def _body(x_ref, wq_ref, k_ref, v_ref, wo_ref, out_ref,
          kg_ref, vg_ref, ar_ref, send_sems, recv_sems):
    me = lax.axis_index("i")

    barrier = pltpu.get_barrier_semaphore()
    for s in range(1, N_DEV):
        pl.semaphore_signal(
            barrier, inc=1,
            device_id=((me + s) % N_DEV,),
            device_id_type=pl.DeviceIdType.MESH,
        )
    pl.semaphore_wait(barrier, N_DEV - 1)

    kg_ref[0] = k_ref[:, :, pl.ds(me * DQ_LOC, DQ_LOC)]
    vg_ref[0] = v_ref[:, :, pl.ds(me * DQ_LOC, DQ_LOC)]

    copies = []
    for s in range(1, N_DEV):
        peer = (me + s) % N_DEV
        kcp = pltpu.make_async_remote_copy(
            src_ref=k_ref.at[:, :, pl.ds(peer * DQ_LOC, DQ_LOC)],
            dst_ref=kg_ref.at[N_DEV - s],
            send_sem=send_sems.at[s - 1],
            recv_sem=recv_sems.at[s - 1],
            device_id=(peer,),
            device_id_type=pl.DeviceIdType.MESH,
        )
        vcp = pltpu.make_async_remote_copy(
            src_ref=v_ref.at[:, :, pl.ds(peer * DQ_LOC, DQ_LOC)],
            dst_ref=vg_ref.at[N_DEV - s],
            send_sem=send_sems.at[3 + s - 1],
            recv_sem=recv_sems.at[3 + s - 1],
            device_id=(peer,),
            device_id_type=pl.DeviceIdType.MESH,
        )
        kcp.start()
        vcp.start()
        copies.append(kcp)
        copies.append(vcp)

    x2d = x_ref[...].reshape(B * SQ, DM)
    qf = lax.dot_general(
        x2d, wq_ref[...], (((1,), (0,)), ((), ())),
        preferred_element_type=jnp.float32,
    )
    qb = qf.astype(jnp.bfloat16).reshape(B, SQ, H_LOC, DH)

    qi = lax.broadcasted_iota(jnp.int32, (SQ, SKV_G), 0)
    col = lax.broadcasted_iota(jnp.int32, (SQ, SKV_G), 1)
    ki = ((me + col // SKV_LOC) % N_DEV) * SKV_LOC + col % SKV_LOC
    mask = (jnp.abs(qi - ki) <= 128) | (ki < 32) | (qi < 32)
    neg = jnp.float32(-1e9)

    for cp in copies:
        cp.wait_recv()

    ctx_rows = []
    for b in range(B):
        ctx_h = []
        for h in range(H_LOC):
            k2d = jnp.concatenate(
                [kg_ref[t, b, :, h * DH:(h + 1) * DH] for t in range(N_DEV)],
                axis=0)
            v2d = jnp.concatenate(
                [vg_ref[t, b, :, h * DH:(h + 1) * DH] for t in range(N_DEV)],
                axis=0)
            scores = lax.dot_general(
                qb[b, :, h, :], k2d, (((1,), (1,)), ((), ())),
                preferred_element_type=jnp.float32,
            ) * 0.125
            scores = jnp.where(mask, scores, neg)
            m = jnp.max(scores, axis=1, keepdims=True)
            w = jnp.exp(scores - m)
            w = w / jnp.sum(w, axis=1, keepdims=True)
            ctx_h.append(lax.dot_general(
                w.astype(jnp.bfloat16), v2d, (((1,), (0,)), ((), ())),
                preferred_element_type=jnp.float32,
            ))
        ctx_rows.append(jnp.concatenate(ctx_h, axis=1))
    ctx2d = jnp.concatenate(ctx_rows, axis=0).astype(jnp.bfloat16)

    partial = lax.dot_general(
        ctx2d, wo_ref[...], (((1,), (0,)), ((), ())),
        preferred_element_type=jnp.float32,
    )
    ar_ref[0] = partial.astype(jnp.bfloat16)

    ar_copies = []
    for s in range(1, N_DEV):
        peer = (me + s) % N_DEV
        cp = pltpu.make_async_remote_copy(
            src_ref=ar_ref.at[0],
            dst_ref=ar_ref.at[N_DEV - s],
            send_sem=send_sems.at[6 + s - 1],
            recv_sem=recv_sems.at[6 + s - 1],
            device_id=(peer,),
            device_id_type=pl.DeviceIdType.MESH,
        )
        cp.start()
        ar_copies.append(cp)
    for cp in ar_copies:
        cp.wait_recv()

    acc = ar_ref[0].astype(jnp.float32)
    for t in range(1, N_DEV):
        acc = acc + ar_ref[t].astype(jnp.float32)
    out_ref[...] = acc.reshape(B, SQ, DM)

    for cp in copies:
        cp.wait_send()
    for cp in ar_copies:
        cp.wait_send()


def kernel(x, Wq, K_ext, V_ext, Wo):
    xb = x.astype(jnp.bfloat16)
    wqb = Wq.astype(jnp.bfloat16)
    wob = Wo.astype(jnp.bfloat16)
    kb = K_ext.reshape(B, SKV_LOC, HQ_G * DH).astype(jnp.bfloat16)
    vb = V_ext.reshape(B, SKV_LOC, HQ_G * DH).astype(jnp.bfloat16)

    out_shape = jax.ShapeDtypeStruct((B, SQ, DM), jnp.float32)
    return pl.pallas_call(
        _body,
        out_shape=out_shape,
        in_specs=[pl.BlockSpec(memory_space=pltpu.VMEM)] * 5,
        out_specs=pl.BlockSpec(memory_space=pltpu.VMEM),
        scratch_shapes=[
            pltpu.VMEM((N_DEV, B, SKV_LOC, DQ_LOC), jnp.bfloat16),
            pltpu.VMEM((N_DEV, B, SKV_LOC, DQ_LOC), jnp.bfloat16),
            pltpu.VMEM((N_DEV, B * SQ, DM), jnp.bfloat16),
            pltpu.SemaphoreType.DMA((9,)),
            pltpu.SemaphoreType.DMA((9,)),
        ],
        compiler_params=pltpu.CompilerParams(collective_id=0),
    )(xb, wqb, kb, vb, wob)


# device time: 132821 ns/iter; 1.0418x vs baseline; 1.0418x over previous
import jax
import jax.numpy as jnp
from jax import lax
from jax.experimental import pallas as pl
from jax.experimental.pallas import tpu as pltpu

N_DEV = 4
B, SQ, SKV_G, HQ_G, DH = 2, 512, 2048, 32, 64
H_LOC = HQ_G // N_DEV
SKV_LOC = SKV_G // N_DEV
DM = 768
DQ_LOC = H_LOC * DH


def _body(x_ref, wq_ref, k_ref, v_ref, wo_ref, out_ref,
          kg_ref, vg_ref, ar_ref, q_ref, ctx_ref, send_sems, recv_sems):
    me = lax.axis_index("i")

    barrier = pltpu.get_barrier_semaphore()
    for s in range(1, N_DEV):
        pl.semaphore_signal(
            barrier, inc=1,
            device_id=((me + s) % N_DEV,),
            device_id_type=pl.DeviceIdType.MESH,
        )
    pl.semaphore_wait(barrier, N_DEV - 1)

    kg_ref[0] = k_ref[:, :, pl.ds(me * DQ_LOC, DQ_LOC)]
    vg_ref[0] = v_ref[:, :, pl.ds(me * DQ_LOC, DQ_LOC)]

    copies = []
    for s in range(1, N_DEV):
        peer = (me + s) % N_DEV
        kcp = pltpu.make_async_remote_copy(
            src_ref=k_ref.at[:, :, pl.ds(peer * DQ_LOC, DQ_LOC)],
            dst_ref=kg_ref.at[N_DEV - s],
            send_sem=send_sems.at[s - 1],
            recv_sem=recv_sems.at[s - 1],
            device_id=(peer,),
            device_id_type=pl.DeviceIdType.MESH,
        )
        vcp = pltpu.make_async_remote_copy(
            src_ref=v_ref.at[:, :, pl.ds(peer * DQ_LOC, DQ_LOC)],
            dst_ref=vg_ref.at[N_DEV - s],
            send_sem=send_sems.at[3 + s - 1],
            recv_sem=recv_sems.at[3 + s - 1],
            device_id=(peer,),
            device_id_type=pl.DeviceIdType.MESH,
        )
        kcp.start()
        vcp.start()
        copies.append(kcp)
        copies.append(vcp)

    x2d = x_ref[...].reshape(B * SQ, DM)
    qf = lax.dot_general(
        x2d, wq_ref[...], (((1,), (0,)), ((), ())),
        preferred_element_type=jnp.float32,
    )
    q_ref[...] = (qf * 0.125).astype(jnp.bfloat16)

    NA = SKV_LOC + 128
    NB = SKV_G - NA
    GR = 32
    qi = lax.broadcasted_iota(jnp.int32, (SQ, NA), 0)
    ki = lax.broadcasted_iota(jnp.int32, (SQ, NA), 1)
    mask_a = (jnp.abs(qi - ki) <= 128) | (ki < 32) | (qi < 32)
    neg = jnp.float32(-1e9)
    minf = jnp.full((SQ - GR, 1), -1e30, jnp.float32)
    zcol = jnp.zeros((SQ - GR, 1), jnp.float32)
    zctx = jnp.zeros((SQ - GR, DH), jnp.float32)

    for cp in copies:
        cp.wait_recv()

    slot = [(jnp.int32(c) - me) % N_DEV for c in range(N_DEV)]

    def dot_t(a, bm):
        return lax.dot_general(a, bm, (((1,), (1,)), ((), ())),
                               preferred_element_type=jnp.float32)

    def dot_n(a, bm):
        return lax.dot_general(a, bm, (((1,), (0,)), ((), ())),
                               preferred_element_type=jnp.float32)

    for b in range(B):
        for h in range(H_LOC):
            hs = slice(h * DH, (h + 1) * DH)
            qh = q_ref[pl.ds(b * SQ, SQ), hs]
            q32 = qh[:GR]
            kc = [kg_ref[pl.ds(slot[c], 1), b, :, hs][0] for c in range(N_DEV)]
            vc = [vg_ref[pl.ds(slot[c], 1), b, :, hs][0] for c in range(N_DEV)]
            sa = jnp.concatenate(
                [dot_t(qh, kc[0]), dot_t(qh, kc[1][:128])], axis=1)
            sa = jnp.where(mask_a, sa, neg)
            sb = jnp.concatenate(
                [dot_t(q32, kc[1][128:]), dot_t(q32, kc[2]),
                 dot_t(q32, kc[3])], axis=1)
            ma = jnp.max(sa, axis=1, keepdims=True)
            mb = jnp.max(sb, axis=1, keepdims=True)
            m = jnp.maximum(ma, jnp.concatenate([mb, minf], axis=0))
            ea = jnp.exp(sa - m).astype(jnp.bfloat16)
            eb = jnp.exp(sb - m[:GR]).astype(jnp.bfloat16)
            denom = (jnp.sum(ea.astype(jnp.float32), axis=1, keepdims=True)
                     + jnp.concatenate(
                         [jnp.sum(eb.astype(jnp.float32), axis=1,
                                  keepdims=True), zcol], axis=0))
            na = dot_n(ea[:, :SKV_LOC], vc[0]) + dot_n(ea[:, SKV_LOC:],
                                                       vc[1][:128])
            nb = (dot_n(eb[:, :SKV_LOC - 128], vc[1][128:])
                  + dot_n(eb[:, SKV_LOC - 128:2 * SKV_LOC - 128], vc[2])
                  + dot_n(eb[:, 2 * SKV_LOC - 128:], vc[3]))
            num = na + jnp.concatenate([nb, zctx], axis=0)
            ctx_ref[pl.ds(b * SQ, SQ), hs] = (num / denom).astype(jnp.bfloat16)
    ctx2d = ctx_ref[...]

    partial = lax.dot_general(
        ctx2d, wo_ref[...], (((1,), (0,)), ((), ())),
        preferred_element_type=jnp.float32,
    )
    ar_ref[0] = partial.astype(jnp.bfloat16)

    ar_copies = []
    for s in range(1, N_DEV):
        peer = (me + s) % N_DEV
        cp = pltpu.make_async_remote_copy(
            src_ref=ar_ref.at[0],
            dst_ref=ar_ref.at[N_DEV - s],
            send_sem=send_sems.at[6 + s - 1],
            recv_sem=recv_sems.at[6 + s - 1],
            device_id=(peer,),
            device_id_type=pl.DeviceIdType.MESH,
        )
        cp.start()
        ar_copies.append(cp)
    for cp in ar_copies:
        cp.wait_recv()

    out_ref[...] = (ar_ref[0].astype(jnp.float32)
                    + ar_ref[1].astype(jnp.float32)).reshape(B, SQ, DM)
    out_ref[...] = (out_ref[...].reshape(B * SQ, DM)
                    + ar_ref[2].astype(jnp.float32)
                    + ar_ref[3].astype(jnp.float32)).reshape(B, SQ, DM)

    for cp in copies:
        cp.wait_send()
    for cp in ar_copies:
        cp.wait_send()


def kernel(x, Wq, K_ext, V_ext, Wo):
    xb = x.astype(jnp.bfloat16)
    wqb = Wq.astype(jnp.bfloat16)
    wob = Wo.astype(jnp.bfloat16)
    kb = K_ext.reshape(B, SKV_LOC, HQ_G * DH).astype(jnp.bfloat16)
    vb = V_ext.reshape(B, SKV_LOC, HQ_G * DH).astype(jnp.bfloat16)

    out_shape = jax.ShapeDtypeStruct((B, SQ, DM), jnp.float32)
    return pl.pallas_call(
        _body,
        out_shape=out_shape,
        in_specs=[pl.BlockSpec(memory_space=pltpu.VMEM)] * 5,
        out_specs=pl.BlockSpec(memory_space=pltpu.VMEM),
        scratch_shapes=[
            pltpu.VMEM((N_DEV, B, SKV_LOC, DQ_LOC), jnp.bfloat16),
            pltpu.VMEM((N_DEV, B, SKV_LOC, DQ_LOC), jnp.bfloat16),
            pltpu.VMEM((N_DEV, B * SQ, DM), jnp.bfloat16),
            pltpu.VMEM((B * SQ, DQ_LOC), jnp.bfloat16),
            pltpu.VMEM((B * SQ, DQ_LOC), jnp.bfloat16),
            pltpu.SemaphoreType.DMA((9,)),
            pltpu.SemaphoreType.DMA((9,)),
        ],
        compiler_params=pltpu.CompilerParams(collective_id=0),
    )(xb, wqb, kb, vb, wob)
